# baseline (speedup 1.0000x reference)
import functools

import jax
import jax.numpy as jnp
from jax import lax
from jax.experimental import pallas as pl
from jax.experimental.pallas import tpu as pltpu
from jax.experimental.pallas import tpu_sc as plsc

_NC = 2
_NS = 16
_NW = _NC * _NS
_CHUNK = 128
_K = 2
_BLOCK_M = 4096
_L = 16


def _pack_chunk(rows_v, pk_v, d):
    rnd = jnp.full((_L,), 0x8000, jnp.int32)
    s16 = jnp.full((_L,), 16, jnp.int32)
    himask = jnp.full((_L,), -65536, jnp.int32)
    n_grp = d // (2 * _L)

    def row_body(r, carry):
        for g in range(n_grp):
            a = rows_v[r, pl.ds(2 * _L * g, _L)]
            bb = rows_v[r, pl.ds(2 * _L * g + _L, _L)]
            ai = lax.bitcast_convert_type(a, jnp.int32)
            bi = lax.bitcast_convert_type(bb, jnp.int32)
            lo = lax.shift_right_logical(ai + rnd, s16)
            hi = (bi + rnd) & himask
            pk_v[r, pl.ds(_L * g, _L)] = lo | hi
        return carry

    lax.fori_loop(0, rows_v.shape[0], row_body, 0)


def _gather_body(table_hbm, idx_hbm, out_hbm, idx_v, rows0_v, rows1_v,
                 pk0_v, pk1_v, sem0, sem1):
    n_chunks = idx_hbm.shape[1]
    d = table_hbm.shape[1]
    wid = lax.axis_index("s") * _NC + lax.axis_index("c")
    base = wid * (n_chunks * _CHUNK)
    pltpu.sync_copy(idx_hbm.at[wid], idx_v)

    def start(j, buf, sem):
        pltpu.make_async_copy(table_hbm.at[idx_v.at[j]], buf, sem).start()

    def wait(buf, sem):
        pltpu.make_async_copy(table_hbm.at[idx_v.at[0]], buf, sem).wait()

    def store(j, pk):
        pltpu.sync_copy(pk, out_hbm.at[pl.ds(base + j * _CHUNK, _CHUNK)])

    start(0, rows0_v, sem0)
    start(1, rows1_v, sem1)

    def step(j2, carry):
        j = 2 * j2
        wait(rows0_v, sem0)
        _pack_chunk(rows0_v, pk0_v, d)

        @pl.when(j + 2 < n_chunks)
        def _():
            start(j + 2, rows0_v, sem0)

        store(j, pk0_v)
        wait(rows1_v, sem1)
        _pack_chunk(rows1_v, pk1_v, d)

        @pl.when(j + 3 < n_chunks)
        def _():
            start(j + 3, rows1_v, sem1)

        store(j + 1, pk1_v)
        return carry

    lax.fori_loop(0, n_chunks // 2, step, 0)


def _sc_gather_pack(table, idx_grouped):
    nw, n_chunks, chunk = idx_grouped.shape
    d = table.shape[1]
    mesh = plsc.VectorSubcoreMesh(core_axis_name="c", subcore_axis_name="s")
    return pl.kernel(
        _gather_body,
        out_type=jax.ShapeDtypeStruct((nw * n_chunks * chunk, d // 2),
                                      jnp.int32),
        mesh=mesh,
        scratch_types=[
            pltpu.VMEM((n_chunks, chunk), jnp.int32),
            pltpu.VMEM((chunk, d), table.dtype),
            pltpu.VMEM((chunk, d), table.dtype),
            pltpu.VMEM((chunk, d // 2), jnp.int32),
            pltpu.VMEM((chunk, d // 2), jnp.int32),
            pltpu.SemaphoreType.DMA,
            pltpu.SemaphoreType.DMA,
        ],
    )(table, idx_grouped)


def _proj_body(x_ref, we_ref, wo_ref, b_ref, o_ref):
    x = x_ref[...]
    a = lax.bitcast_convert_type(lax.shift_left(x, 16), jnp.float32)
    c = lax.bitcast_convert_type(x & jnp.int32(-65536), jnp.float32)
    acc = lax.dot_general(
        a, we_ref[...],
        dimension_numbers=(((1,), (1,)), ((), ())),
        preferred_element_type=jnp.float32,
    )
    acc += lax.dot_general(
        c, wo_ref[...],
        dimension_numbers=(((1,), (1,)), ((), ())),
        preferred_element_type=jnp.float32,
    )
    o_ref[...] = acc + b_ref[...]


def _proj_body_aliased(x_ref, we_ref, wo_ref, b_ref, prev_ref, o_ref):
    del prev_ref
    _proj_body(x_ref, we_ref, wo_ref, b_ref, o_ref)


def _tc_project_slice(emb_k, we, wo, b, prev, k0_blocks, n):
    m, d2 = emb_k.shape
    h = we.shape[0]
    grid = (m // _BLOCK_M,)
    in_specs = [
        pl.BlockSpec((_BLOCK_M, d2), lambda i: (i, 0)),
        pl.BlockSpec((h, d2), lambda i: (0, 0)),
        pl.BlockSpec((h, d2), lambda i: (0, 0)),
        pl.BlockSpec((1, h), lambda i: (0, 0)),
    ]
    args = [emb_k, we, wo, b]
    body = _proj_body
    aliases = {}
    if prev is not None:
        in_specs.append(pl.BlockSpec(memory_space=pl.ANY))
        args.append(prev)
        body = _proj_body_aliased
        aliases = {4: 0}
    return pl.pallas_call(
        body,
        grid=grid,
        in_specs=in_specs,
        out_specs=pl.BlockSpec((_BLOCK_M, h),
                               lambda i, k0=k0_blocks: (k0 + i, 0)),
        out_shape=jax.ShapeDtypeStruct((n, h), jnp.float32),
        input_output_aliases=aliases,
    )(*args)


def kernel(input, embedding_matrix, W, b):
    bsz, seq = input.shape
    n_tok = bsz * seq
    h = W.shape[0]
    idx = input.reshape(_K, _NW, n_tok // (_K * _NW * _CHUNK), _CHUNK)
    idx = idx.astype(jnp.int32)
    b2 = b.reshape(1, -1)
    d = embedding_matrix.shape[1]
    perm_lo = jnp.array([32 * (w // 16) + w % 16 for w in range(d // 2)],
                        dtype=jnp.int32)
    we = jnp.take(W, perm_lo, axis=1)
    wo = jnp.take(W, perm_lo + 16, axis=1)

    embs = [_sc_gather_pack(embedding_matrix, idx[k]) for k in range(_K)]

    m = n_tok // _K
    stripe_blocks = m // _BLOCK_M
    out = None
    for k in range(_K):
        out = _tc_project_slice(embs[k], we, wo, b2, out,
                                k * stripe_blocks, n_tok)
    return out.reshape(bsz, seq, h)


# speedup vs baseline: 1.0074x; 1.0074x over previous
import functools

import jax
import jax.numpy as jnp
from jax import lax
from jax.experimental import pallas as pl
from jax.experimental.pallas import tpu as pltpu
from jax.experimental.pallas import tpu_sc as plsc

_NC = 2
_NS = 16
_NW = _NC * _NS
_CHUNK = 128
_K = 1
_BLOCK_M = 4096
_L = 16


def _pack_chunk(rows_v, pk_v, d):
    rnd = jnp.full((_L,), 0x8000, jnp.int32)
    s16 = jnp.full((_L,), 16, jnp.int32)
    himask = jnp.full((_L,), -65536, jnp.int32)
    n_grp = d // (2 * _L)

    def row_body(r, carry):
        for g in range(n_grp):
            a = rows_v[r, pl.ds(2 * _L * g, _L)]
            bb = rows_v[r, pl.ds(2 * _L * g + _L, _L)]
            ai = lax.bitcast_convert_type(a, jnp.int32)
            bi = lax.bitcast_convert_type(bb, jnp.int32)
            lo = lax.shift_right_logical(ai + rnd, s16)
            hi = (bi + rnd) & himask
            pk_v[r, pl.ds(_L * g, _L)] = lo | hi
        return carry

    lax.fori_loop(0, rows_v.shape[0], row_body, 0)


def _gather_body(table_hbm, idx_hbm, out_hbm, idx_v, rows0_v, rows1_v,
                 pk0_v, pk1_v, sem0, sem1):
    n_chunks = idx_hbm.shape[1]
    d = table_hbm.shape[1]
    wid = lax.axis_index("s") * _NC + lax.axis_index("c")
    base = wid * (n_chunks * _CHUNK)
    pltpu.sync_copy(idx_hbm.at[wid], idx_v)

    def start(j, buf, sem):
        pltpu.make_async_copy(table_hbm.at[idx_v.at[j]], buf, sem).start()

    def wait(buf, sem):
        pltpu.make_async_copy(table_hbm.at[idx_v.at[0]], buf, sem).wait()

    def store(j, pk):
        pltpu.sync_copy(pk, out_hbm.at[pl.ds(base + j * _CHUNK, _CHUNK)])

    start(0, rows0_v, sem0)
    start(1, rows1_v, sem1)

    def step(j2, carry):
        j = 2 * j2
        wait(rows0_v, sem0)
        _pack_chunk(rows0_v, pk0_v, d)

        @pl.when(j + 2 < n_chunks)
        def _():
            start(j + 2, rows0_v, sem0)

        store(j, pk0_v)
        wait(rows1_v, sem1)
        _pack_chunk(rows1_v, pk1_v, d)

        @pl.when(j + 3 < n_chunks)
        def _():
            start(j + 3, rows1_v, sem1)

        store(j + 1, pk1_v)
        return carry

    lax.fori_loop(0, n_chunks // 2, step, 0)


def _sc_gather_pack(table, idx_grouped):
    nw, n_chunks, chunk = idx_grouped.shape
    d = table.shape[1]
    mesh = plsc.VectorSubcoreMesh(core_axis_name="c", subcore_axis_name="s")
    return pl.kernel(
        _gather_body,
        out_type=jax.ShapeDtypeStruct((nw * n_chunks * chunk, d // 2),
                                      jnp.int32),
        mesh=mesh,
        scratch_types=[
            pltpu.VMEM((n_chunks, chunk), jnp.int32),
            pltpu.VMEM((chunk, d), table.dtype),
            pltpu.VMEM((chunk, d), table.dtype),
            pltpu.VMEM((chunk, d // 2), jnp.int32),
            pltpu.VMEM((chunk, d // 2), jnp.int32),
            pltpu.SemaphoreType.DMA,
            pltpu.SemaphoreType.DMA,
        ],
    )(table, idx_grouped)


def _proj_body(x_ref, we_ref, wo_ref, b_ref, o_ref):
    x = x_ref[...]
    a = lax.bitcast_convert_type(lax.shift_left(x, 16), jnp.float32)
    c = lax.bitcast_convert_type(x & jnp.int32(-65536), jnp.float32)
    acc = lax.dot_general(
        a, we_ref[...],
        dimension_numbers=(((1,), (1,)), ((), ())),
        preferred_element_type=jnp.float32,
    )
    acc += lax.dot_general(
        c, wo_ref[...],
        dimension_numbers=(((1,), (1,)), ((), ())),
        preferred_element_type=jnp.float32,
    )
    o_ref[...] = acc + b_ref[...]


def _proj_body_aliased(x_ref, we_ref, wo_ref, b_ref, prev_ref, o_ref):
    del prev_ref
    _proj_body(x_ref, we_ref, wo_ref, b_ref, o_ref)


def _tc_project_slice(emb_k, we, wo, b, prev, k0_blocks, n):
    m, d2 = emb_k.shape
    h = we.shape[0]
    grid = (m // _BLOCK_M,)
    in_specs = [
        pl.BlockSpec((_BLOCK_M, d2), lambda i: (i, 0)),
        pl.BlockSpec((h, d2), lambda i: (0, 0)),
        pl.BlockSpec((h, d2), lambda i: (0, 0)),
        pl.BlockSpec((1, h), lambda i: (0, 0)),
    ]
    args = [emb_k, we, wo, b]
    body = _proj_body
    aliases = {}
    if prev is not None:
        in_specs.append(pl.BlockSpec(memory_space=pl.ANY))
        args.append(prev)
        body = _proj_body_aliased
        aliases = {4: 0}
    return pl.pallas_call(
        body,
        grid=grid,
        in_specs=in_specs,
        out_specs=pl.BlockSpec((_BLOCK_M, h),
                               lambda i, k0=k0_blocks: (k0 + i, 0)),
        out_shape=jax.ShapeDtypeStruct((n, h), jnp.float32),
        input_output_aliases=aliases,
    )(*args)


def kernel(input, embedding_matrix, W, b):
    bsz, seq = input.shape
    n_tok = bsz * seq
    h = W.shape[0]
    idx = input.reshape(_K, _NW, n_tok // (_K * _NW * _CHUNK), _CHUNK)
    idx = idx.astype(jnp.int32)
    b2 = b.reshape(1, -1)
    d = embedding_matrix.shape[1]
    perm_lo = jnp.array([32 * (w // 16) + w % 16 for w in range(d // 2)],
                        dtype=jnp.int32)
    we = jnp.take(W, perm_lo, axis=1)
    wo = jnp.take(W, perm_lo + 16, axis=1)

    embs = [_sc_gather_pack(embedding_matrix, idx[k]) for k in range(_K)]

    m = n_tok // _K
    stripe_blocks = m // _BLOCK_M
    out = None
    for k in range(_K):
        out = _tc_project_slice(embs[k], we, wo, b2, out,
                                k * stripe_blocks, n_tok)
    return out.reshape(bsz, seq, h)


# speedup vs baseline: 1.0120x; 1.0045x over previous
import functools

import jax
import jax.numpy as jnp
from jax import lax
from jax.experimental import pallas as pl
from jax.experimental.pallas import tpu as pltpu
from jax.experimental.pallas import tpu_sc as plsc

_NC = 2
_NS = 16
_NW = _NC * _NS
_CHUNK = 128
_K = 1
_BLOCK_M = 4096
_L = 16


def _pack_chunk(rows_v, pk_v, d):
    rnd = jnp.full((_L,), 0x8000, jnp.int32)
    s16 = jnp.full((_L,), 16, jnp.int32)
    himask = jnp.full((_L,), -65536, jnp.int32)
    n_grp = d // (2 * _L)

    def row_body(r, carry):
        for g in range(n_grp):
            a = rows_v[r, pl.ds(2 * _L * g, _L)]
            bb = rows_v[r, pl.ds(2 * _L * g + _L, _L)]
            ai = lax.bitcast_convert_type(a, jnp.int32)
            bi = lax.bitcast_convert_type(bb, jnp.int32)
            lo = lax.shift_right_logical(ai + rnd, s16)
            hi = (bi + rnd) & himask
            pk_v[r, pl.ds(_L * g, _L)] = lo | hi
        return carry

    lax.fori_loop(0, rows_v.shape[0], row_body, 0)


def _gather_body(table_hbm, idx_hbm, out_hbm, idx_v, rows0_v, rows1_v,
                 pk0_v, pk1_v, sem0, sem1):
    n_chunks = idx_hbm.shape[1]
    d = table_hbm.shape[1]
    wid = lax.axis_index("s") * _NC + lax.axis_index("c")
    base = wid * (n_chunks * _CHUNK)
    pltpu.sync_copy(idx_hbm.at[wid], idx_v)

    def start(j, buf, sem):
        pltpu.make_async_copy(table_hbm.at[idx_v.at[j]], buf, sem).start()

    def wait(buf, sem):
        pltpu.make_async_copy(table_hbm.at[idx_v.at[0]], buf, sem).wait()

    def store(j, pk):
        pltpu.sync_copy(pk, out_hbm.at[pl.ds(base + j * _CHUNK, _CHUNK)])

    start(0, rows0_v, sem0)
    start(1, rows1_v, sem1)

    def step(j2, carry):
        j = 2 * j2
        wait(rows0_v, sem0)
        _pack_chunk(rows0_v, pk0_v, d)

        @pl.when(j + 2 < n_chunks)
        def _():
            start(j + 2, rows0_v, sem0)

        store(j, pk0_v)
        wait(rows1_v, sem1)
        _pack_chunk(rows1_v, pk1_v, d)

        @pl.when(j + 3 < n_chunks)
        def _():
            start(j + 3, rows1_v, sem1)

        store(j + 1, pk1_v)
        return carry

    lax.fori_loop(0, n_chunks // 2, step, 0)


def _sc_gather_pack(table, idx_grouped):
    nw, n_chunks, chunk = idx_grouped.shape
    d = table.shape[1]
    mesh = plsc.VectorSubcoreMesh(core_axis_name="c", subcore_axis_name="s")
    return pl.kernel(
        _gather_body,
        out_type=jax.ShapeDtypeStruct((nw * n_chunks * chunk, d // 2),
                                      jnp.int32),
        mesh=mesh,
        scratch_types=[
            pltpu.VMEM((n_chunks, chunk), jnp.int32),
            pltpu.VMEM((chunk, d), table.dtype),
            pltpu.VMEM((chunk, d), table.dtype),
            pltpu.VMEM((chunk, d // 2), jnp.int32),
            pltpu.VMEM((chunk, d // 2), jnp.int32),
            pltpu.SemaphoreType.DMA,
            pltpu.SemaphoreType.DMA,
        ],
    )(table, idx_grouped)


def _proj_body(x_ref, w_ref, b_ref, o_ref):
    x = x_ref[...]
    a = lax.bitcast_convert_type(lax.shift_left(x, 16), jnp.float32)
    c = lax.bitcast_convert_type(x & jnp.int32(-65536), jnp.float32)
    xf = jnp.concatenate([a, c], axis=1)
    acc = lax.dot_general(
        xf, w_ref[...],
        dimension_numbers=(((1,), (1,)), ((), ())),
        preferred_element_type=jnp.float32,
    )
    o_ref[...] = acc + b_ref[...]


def _proj_body_aliased(x_ref, w_ref, b_ref, prev_ref, o_ref):
    del prev_ref
    _proj_body(x_ref, w_ref, b_ref, o_ref)


def _tc_project_slice(emb_k, w2, b, prev, k0_blocks, n):
    m, d2 = emb_k.shape
    h = w2.shape[0]
    grid = (m // _BLOCK_M,)
    in_specs = [
        pl.BlockSpec((_BLOCK_M, d2), lambda i: (i, 0)),
        pl.BlockSpec((h, 2 * d2), lambda i: (0, 0)),
        pl.BlockSpec((1, h), lambda i: (0, 0)),
    ]
    args = [emb_k, w2, b]
    body = _proj_body
    aliases = {}
    if prev is not None:
        in_specs.append(pl.BlockSpec(memory_space=pl.ANY))
        args.append(prev)
        body = _proj_body_aliased
        aliases = {3: 0}
    return pl.pallas_call(
        body,
        grid=grid,
        in_specs=in_specs,
        out_specs=pl.BlockSpec((_BLOCK_M, h),
                               lambda i, k0=k0_blocks: (k0 + i, 0)),
        out_shape=jax.ShapeDtypeStruct((n, h), jnp.float32),
        input_output_aliases=aliases,
    )(*args)


def kernel(input, embedding_matrix, W, b):
    bsz, seq = input.shape
    n_tok = bsz * seq
    h = W.shape[0]
    idx = input.reshape(_K, _NW, n_tok // (_K * _NW * _CHUNK), _CHUNK)
    idx = idx.astype(jnp.int32)
    b2 = b.reshape(1, -1)
    d = embedding_matrix.shape[1]
    perm_lo = jnp.array([32 * (w // 16) + w % 16 for w in range(d // 2)],
                        dtype=jnp.int32)
    w2 = jnp.concatenate(
        [jnp.take(W, perm_lo, axis=1), jnp.take(W, perm_lo + 16, axis=1)],
        axis=1)

    embs = [_sc_gather_pack(embedding_matrix, idx[k]) for k in range(_K)]

    m = n_tok // _K
    stripe_blocks = m // _BLOCK_M
    out = None
    for k in range(_K):
        out = _tc_project_slice(embs[k], w2, b2, out,
                                k * stripe_blocks, n_tok)
    return out.reshape(bsz, seq, h)
